# baseline (speedup 1.0000x reference)
import functools

import jax
import jax.numpy as jnp
from jax import lax
from jax.experimental import pallas as pl
from jax.experimental.pallas import tpu as pltpu
from jax.experimental.pallas import tpu_sc as plsc

BATCH = 4096
T = 200
D = 64
LANES = 16

NC = 2
NS = 16
NW = NC * NS

FLAT = BATCH * T
ROWS_PER_W = FLAT // NW
SUB = 128
NSUB = 4
CHUNK = SUB * NSUB
NCHUNKS = ROWS_PER_W // CHUNK

_mesh = plsc.VectorSubcoreMesh(core_axis_name="c", subcore_axis_name="s")


@functools.partial(
    pl.kernel,
    out_type=jax.ShapeDtypeStruct((NW * NCHUNKS, NSUB, SUB, D), jnp.float32),
    mesh=_mesh,
    compiler_params=pltpu.CompilerParams(use_tc_tiling_on_sc=False),
    scratch_types=[
        pltpu.VMEM((NSUB, SUB), jnp.int32),
        pltpu.VMEM((NSUB, SUB, D), jnp.float32),
        pltpu.VMEM((T, D), jnp.float32),
        pltpu.SemaphoreType.DMA,
    ],
)
def _emb_lookup(x_hbm, tab_hbm, pemb_hbm, out_hbm, idx_v, rows_v, pemb_v, sem):
    wid = lax.axis_index("s") * NC + lax.axis_index("c")
    pltpu.sync_copy(pemb_hbm, pemb_v)

    def do_chunk(c, carry):
        g = wid * NCHUNKS + c
        pltpu.sync_copy(x_hbm.at[g], idx_v)
        copies = [
            pltpu.async_copy(tab_hbm.at[idx_v.at[j]], rows_v.at[j], sem)
            for j in range(NSUB)
        ]
        for cp in copies:
            cp.wait()
        base = c * CHUNK
        for j in range(NSUB):
            off_j = base + j * SUB

            def add_row(i, _, off_j=off_j, j=j):
                t = lax.rem(off_j + i, T)
                for k in range(D // LANES):
                    sl = pl.ds(k * LANES, LANES)
                    rows_v[j, i, sl] = rows_v[j, i, sl] + pemb_v[t, sl]
                return 0

            lax.fori_loop(0, SUB, add_row, 0)
        pltpu.sync_copy(rows_v, out_hbm.at[g])
        return carry

    lax.fori_loop(0, NCHUNKS, do_chunk, 0)


def kernel(x, t_emb, p_emb):
    xr = x.reshape(NW * NCHUNKS, NSUB, SUB).astype(jnp.int32)
    out = _emb_lookup(xr, t_emb, p_emb)
    return out.reshape(BATCH, T, D)


# speedup vs baseline: 1.0597x; 1.0597x over previous
import functools

import jax
import jax.numpy as jnp
from jax import lax
from jax.experimental import pallas as pl
from jax.experimental.pallas import tpu as pltpu
from jax.experimental.pallas import tpu_sc as plsc

BATCH = 4096
T = 200
D = 64
LANES = 16

NC = 2
NS = 16
NW = NC * NS

FLAT = BATCH * T
ROWS_PER_W = FLAT // NW
SUB = 128
NSUB = 4
CHUNK = SUB * NSUB
NCHUNKS = ROWS_PER_W // CHUNK

_mesh = plsc.VectorSubcoreMesh(core_axis_name="c", subcore_axis_name="s")


@functools.partial(
    pl.kernel,
    out_type=jax.ShapeDtypeStruct((NW * NCHUNKS, NSUB, SUB, D), jnp.float32),
    mesh=_mesh,
    compiler_params=pltpu.CompilerParams(use_tc_tiling_on_sc=False),
    scratch_types=[
        pltpu.VMEM((2, NSUB, SUB), jnp.int32),
        pltpu.VMEM((2, NSUB, SUB, D), jnp.float32),
        pltpu.VMEM((T, D), jnp.float32),
        pltpu.SemaphoreType.DMA,
        pltpu.SemaphoreType.DMA,
    ],
)
def _emb_lookup(x_hbm, tab_hbm, pemb_hbm, out_hbm, idx_v, rows_v, pemb_v,
                sem0, sem1):
    wid = lax.axis_index("s") * NC + lax.axis_index("c")
    base_chunk = wid * NCHUNKS
    sems = (sem0, sem1)
    pltpu.sync_copy(pemb_hbm, pemb_v)

    def fire(chunk, s):
        pltpu.sync_copy(x_hbm.at[base_chunk + chunk], idx_v.at[s])
        for j in range(NSUB):
            pltpu.make_async_copy(
                tab_hbm.at[idx_v.at[s, j]], rows_v.at[s, j], sems[s]
            ).start()

    def drain(s):
        for j in range(NSUB):
            pltpu.make_async_copy(
                tab_hbm.at[idx_v.at[s, j]], rows_v.at[s, j], sems[s]
            ).wait()

    fire(0, 0)

    def do_pair(c2, carry):
        for s in range(2):
            c = 2 * c2 + s

            @pl.when(c + 1 < NCHUNKS)
            def _():
                fire(c + 1, 1 - s)

            drain(s)
            base = c * CHUNK
            for j in range(NSUB):
                off_j = base + j * SUB

                def add_row(i, _, off_j=off_j, j=j, s=s):
                    t = lax.rem(off_j + i, T)
                    for k in range(D // LANES):
                        sl = pl.ds(k * LANES, LANES)
                        rows_v[s, j, i, sl] = rows_v[s, j, i, sl] + pemb_v[t, sl]
                    return 0

                lax.fori_loop(0, SUB, add_row, 0)
            pltpu.sync_copy(rows_v.at[s], out_hbm.at[base_chunk + c])
        return carry

    lax.fori_loop(0, NCHUNKS // 2, do_pair, 0)


def kernel(x, t_emb, p_emb):
    xr = x.reshape(NW * NCHUNKS, NSUB, SUB).astype(jnp.int32)
    out = _emb_lookup(xr, t_emb, p_emb)
    return out.reshape(BATCH, T, D)
